# baseline (speedup 1.0000x reference)
import functools

import jax
import jax.numpy as jnp
from jax import lax
from jax.experimental import pallas as pl
from jax.experimental.pallas import tpu as pltpu
from jax.experimental.pallas import tpu_sc as plsc

NUM_CORES = 2
NUM_SUBCORES = 16
NUM_WORKERS = NUM_CORES * NUM_SUBCORES
LANES = 16

EMBED_DIM = 64
PAIR_DIM = 2 * EMBED_DIM
CHUNK = 128
N_CHUNKS = 4
ROWS_PER_WORKER = CHUNK * N_CHUNKS
PASS_ROWS = 256
N_PASSES = ROWS_PER_WORKER // PASS_ROWS


def _sc_body(in_tab, out_tab, idx_in, idx_pos, idx_neg,
             par_in, par_pos, par_neg,
             pos_out, neg_out,
             idxv_in, idxv_pos, idxv_neg,
             parv_in, parv_pos, parv_neg,
             rows_in, rows_pos, rows_neg,
             pacc_v, nacc_v,
             score_pos, score_neg, sem):
    wid = lax.axis_index("s") * NUM_CORES + lax.axis_index("c")
    rbase = wid * N_CHUNKS
    base = wid * ROWS_PER_WORKER

    pltpu.sync_copy(idx_in.at[pl.ds(rbase, N_CHUNKS)], idxv_in)
    pltpu.sync_copy(idx_pos.at[pl.ds(rbase, N_CHUNKS)], idxv_pos)
    pltpu.sync_copy(idx_neg.at[pl.ds(rbase, N_CHUNKS)], idxv_neg)
    pltpu.sync_copy(par_in.at[pl.ds(base, ROWS_PER_WORKER)], parv_in)
    pltpu.sync_copy(par_pos.at[pl.ds(base, ROWS_PER_WORKER)], parv_pos)
    pltpu.sync_copy(par_neg.at[pl.ds(base, ROWS_PER_WORKER)], parv_neg)

    iota16 = lax.iota(jnp.int32, LANES)

    for p in range(N_PASSES):
        copies = []
        for j in range(PASS_ROWS // CHUNK):
            cj = p * (PASS_ROWS // CHUNK) + j
            sl = pl.ds(j * CHUNK, CHUNK)
            copies.append(pltpu.async_copy(
                in_tab.at[idxv_in.at[cj]], rows_in.at[sl], sem))
            copies.append(pltpu.async_copy(
                out_tab.at[idxv_pos.at[cj]], rows_pos.at[sl], sem))
            copies.append(pltpu.async_copy(
                out_tab.at[idxv_neg.at[cj]], rows_neg.at[sl], sem))
        for c in copies:
            c.wait()

        def chunk_body(c, carry):
            g0 = p * PASS_ROWS + c * LANES
            pv_in = parv_in[pl.ds(g0, LANES)]
            pv_pos = parv_pos[pl.ds(g0, LANES)]
            pv_neg = parv_neg[pl.ds(g0, LANES)]
            for i in range(LANES):
                r = c * LANES + i
                m_in = lax.broadcast(pv_in[i], (LANES,)) > 0
                m_pos = lax.broadcast(pv_pos[i], (LANES,)) > 0
                m_neg = lax.broadcast(pv_neg[i], (LANES,)) > 0
                accp = jnp.zeros((LANES,), jnp.float32)
                accn = jnp.zeros((LANES,), jnp.float32)
                for k in range(EMBED_DIM // LANES):
                    lo = pl.ds(k * LANES, LANES)
                    hi = pl.ds(EMBED_DIM + k * LANES, LANES)
                    a = jnp.where(m_in, rows_in[r, hi], rows_in[r, lo])
                    pvec = jnp.where(m_pos, rows_pos[r, hi], rows_pos[r, lo])
                    nvec = jnp.where(m_neg, rows_neg[r, hi], rows_neg[r, lo])
                    accp = accp + a * pvec
                    accn = accn + a * nvec
                pacc_v[pl.ds(i * LANES, LANES)] = accp
                nacc_v[pl.ds(i * LANES, LANES)] = accn
            totp = jnp.zeros((LANES,), jnp.float32)
            totn = jnp.zeros((LANES,), jnp.float32)
            for d in range(LANES):
                idx = iota16 * LANES + d
                totp = totp + plsc.load_gather(pacc_v, [idx])
                totn = totn + plsc.load_gather(nacc_v, [idx])
            score_pos[pl.ds(g0, LANES)] = totp
            score_neg[pl.ds(g0, LANES)] = totn
            return carry

        lax.fori_loop(0, PASS_ROWS // LANES, chunk_body, 0)

    pltpu.sync_copy(score_pos, pos_out.at[pl.ds(base, ROWS_PER_WORKER)])
    pltpu.sync_copy(score_neg, neg_out.at[pl.ds(base, ROWS_PER_WORKER)])


@jax.jit
def _skipgram_scores(in_tab, out_tab, idx_in, idx_pos, idx_neg,
                     par_in, par_pos, par_neg):
    batch = par_in.shape[0]
    mesh = plsc.VectorSubcoreMesh(
        core_axis_name="c", subcore_axis_name="s",
        num_cores=NUM_CORES, num_subcores=NUM_SUBCORES)
    run = pl.kernel(
        _sc_body,
        out_type=(
            jax.ShapeDtypeStruct((batch,), jnp.float32),
            jax.ShapeDtypeStruct((batch,), jnp.float32),
        ),
        mesh=mesh,
        scratch_types=[
            pltpu.VMEM((N_CHUNKS, CHUNK), jnp.int32),
            pltpu.VMEM((N_CHUNKS, CHUNK), jnp.int32),
            pltpu.VMEM((N_CHUNKS, CHUNK), jnp.int32),
            pltpu.VMEM((ROWS_PER_WORKER,), jnp.int32),
            pltpu.VMEM((ROWS_PER_WORKER,), jnp.int32),
            pltpu.VMEM((ROWS_PER_WORKER,), jnp.int32),
            pltpu.VMEM((PASS_ROWS, PAIR_DIM), jnp.float32),
            pltpu.VMEM((PASS_ROWS, PAIR_DIM), jnp.float32),
            pltpu.VMEM((PASS_ROWS, PAIR_DIM), jnp.float32),
            pltpu.VMEM((LANES * LANES,), jnp.float32),
            pltpu.VMEM((LANES * LANES,), jnp.float32),
            pltpu.VMEM((ROWS_PER_WORKER,), jnp.float32),
            pltpu.VMEM((ROWS_PER_WORKER,), jnp.float32),
            pltpu.SemaphoreType.DMA,
        ],
        compiler_params=pltpu.CompilerParams(
            needs_layout_passes=False, use_tc_tiling_on_sc=True),
    )
    return run(in_tab, out_tab, idx_in, idx_pos, idx_neg,
               par_in, par_pos, par_neg)


def kernel(input_labels, pos_labels, neg_labels, in_embed, out_embed):
    batch = input_labels.shape[0]
    vocab = in_embed.shape[0]
    in_tab = in_embed.reshape(vocab // 2, PAIR_DIM)
    out_tab = out_embed.reshape(vocab // 2, PAIR_DIM)
    li = input_labels.astype(jnp.int32)
    lp = pos_labels.astype(jnp.int32)
    ln = neg_labels.astype(jnp.int32)
    idx_in = (li // 2).reshape(batch // CHUNK, CHUNK)
    idx_pos = (lp // 2).reshape(batch // CHUNK, CHUNK)
    idx_neg = (ln // 2).reshape(batch // CHUNK, CHUNK)
    par_in = li % 2
    par_pos = lp % 2
    par_neg = ln % 2
    pos_score, neg_score = _skipgram_scores(
        in_tab, out_tab, idx_in, idx_pos, idx_neg, par_in, par_pos, par_neg)
    return pos_score, neg_score.reshape(batch, 1)


# speedup vs baseline: 1.0095x; 1.0095x over previous
import jax
import jax.numpy as jnp
from jax import lax
from jax.experimental import pallas as pl
from jax.experimental.pallas import tpu as pltpu
from jax.experimental.pallas import tpu_sc as plsc

NUM_CORES = 2
NUM_SUBCORES = 16
NUM_WORKERS = NUM_CORES * NUM_SUBCORES
LANES = 16

EMBED_DIM = 64
CHUNK = 128
N_CHUNKS = 4
ROWS_PER_WORKER = CHUNK * N_CHUNKS


def _wid():
    return lax.axis_index("s") * NUM_CORES + lax.axis_index("c")


def _gather_body(tab, idx, out, idxv, rows, sem):
    wid = _wid()
    pltpu.sync_copy(idx.at[pl.ds(wid * N_CHUNKS, N_CHUNKS)], idxv)
    copies = []
    for j in range(N_CHUNKS):
        copies.append(pltpu.async_copy(
            tab.at[idxv.at[j]], rows.at[pl.ds(j * CHUNK, CHUNK)], sem))
    for c in copies:
        c.wait()
    pltpu.sync_copy(rows, out.at[pl.ds(wid * ROWS_PER_WORKER, ROWS_PER_WORKER)])


def _score_body(tab, idx_pos, idx_neg, invec,
                pos_out, neg_out,
                idxv_pos, idxv_neg,
                rows_in, rows_pos, rows_neg,
                pacc_v, nacc_v,
                score_pos, score_neg, sem):
    wid = _wid()
    rbase = wid * N_CHUNKS
    base = wid * ROWS_PER_WORKER

    pltpu.sync_copy(idx_pos.at[pl.ds(rbase, N_CHUNKS)], idxv_pos)
    pltpu.sync_copy(idx_neg.at[pl.ds(rbase, N_CHUNKS)], idxv_neg)

    copies = [pltpu.async_copy(
        invec.at[pl.ds(base, ROWS_PER_WORKER)], rows_in, sem)]
    for j in range(N_CHUNKS):
        sl = pl.ds(j * CHUNK, CHUNK)
        copies.append(pltpu.async_copy(
            tab.at[idxv_pos.at[j]], rows_pos.at[sl], sem))
        copies.append(pltpu.async_copy(
            tab.at[idxv_neg.at[j]], rows_neg.at[sl], sem))
    for c in copies:
        c.wait()

    iota16 = lax.iota(jnp.int32, LANES)

    def chunk_body(c, carry):
        for i in range(LANES):
            r = c * LANES + i
            accp = jnp.zeros((LANES,), jnp.float32)
            accn = jnp.zeros((LANES,), jnp.float32)
            for k in range(EMBED_DIM // LANES):
                sl = pl.ds(k * LANES, LANES)
                a = rows_in[r, sl]
                p = rows_pos[r, sl]
                n = rows_neg[r, sl]
                accp = accp + a * p
                accn = accn + a * n
            pacc_v[pl.ds(i * LANES, LANES)] = accp
            nacc_v[pl.ds(i * LANES, LANES)] = accn
        totp = jnp.zeros((LANES,), jnp.float32)
        totn = jnp.zeros((LANES,), jnp.float32)
        for d in range(LANES):
            idx = iota16 * LANES + d
            totp = totp + plsc.load_gather(pacc_v, [idx])
            totn = totn + plsc.load_gather(nacc_v, [idx])
        score_pos[pl.ds(c * LANES, LANES)] = totp
        score_neg[pl.ds(c * LANES, LANES)] = totn
        return carry

    lax.fori_loop(0, ROWS_PER_WORKER // LANES, chunk_body, 0)

    pltpu.sync_copy(score_pos, pos_out.at[pl.ds(base, ROWS_PER_WORKER)])
    pltpu.sync_copy(score_neg, neg_out.at[pl.ds(base, ROWS_PER_WORKER)])


def _mesh():
    return plsc.VectorSubcoreMesh(
        core_axis_name="c", subcore_axis_name="s",
        num_cores=NUM_CORES, num_subcores=NUM_SUBCORES)


@jax.jit
def _skipgram_scores(in_embed, out_embed, idx_in, idx_pos, idx_neg):
    batch = idx_in.shape[0] * idx_in.shape[1]
    params = pltpu.CompilerParams(
        needs_layout_passes=False, use_tc_tiling_on_sc=False)
    gather_in = pl.kernel(
        _gather_body,
        out_type=jax.ShapeDtypeStruct((batch, EMBED_DIM), jnp.float32),
        mesh=_mesh(),
        scratch_types=[
            pltpu.VMEM((N_CHUNKS, CHUNK), jnp.int32),
            pltpu.VMEM((ROWS_PER_WORKER, EMBED_DIM), jnp.float32),
            pltpu.SemaphoreType.DMA,
        ],
        compiler_params=params,
    )
    invec = gather_in(in_embed, idx_in)
    score = pl.kernel(
        _score_body,
        out_type=(
            jax.ShapeDtypeStruct((batch,), jnp.float32),
            jax.ShapeDtypeStruct((batch,), jnp.float32),
        ),
        mesh=_mesh(),
        scratch_types=[
            pltpu.VMEM((N_CHUNKS, CHUNK), jnp.int32),
            pltpu.VMEM((N_CHUNKS, CHUNK), jnp.int32),
            pltpu.VMEM((ROWS_PER_WORKER, EMBED_DIM), jnp.float32),
            pltpu.VMEM((ROWS_PER_WORKER, EMBED_DIM), jnp.float32),
            pltpu.VMEM((ROWS_PER_WORKER, EMBED_DIM), jnp.float32),
            pltpu.VMEM((LANES * LANES,), jnp.float32),
            pltpu.VMEM((LANES * LANES,), jnp.float32),
            pltpu.VMEM((ROWS_PER_WORKER,), jnp.float32),
            pltpu.VMEM((ROWS_PER_WORKER,), jnp.float32),
            pltpu.SemaphoreType.DMA,
        ],
        compiler_params=params,
    )
    return score(out_embed, idx_pos, idx_neg, invec)


def kernel(input_labels, pos_labels, neg_labels, in_embed, out_embed):
    batch = input_labels.shape[0]
    idx_in = input_labels.astype(jnp.int32).reshape(batch // CHUNK, CHUNK)
    idx_pos = pos_labels.astype(jnp.int32).reshape(batch // CHUNK, CHUNK)
    idx_neg = neg_labels.astype(jnp.int32).reshape(batch // CHUNK, CHUNK)
    pos_score, neg_score = _skipgram_scores(
        in_embed, out_embed, idx_in, idx_pos, idx_neg)
    return pos_score, neg_score.reshape(batch, 1)
